# Initial kernel scaffold; baseline (speedup 1.0000x reference)
import jax
import jax.numpy as jnp
from jax.experimental import pallas as pl


def kernel(points, params):
    raise NotImplementedError("write your pallas kernel here")


# speedup vs baseline: 1.3104x; 1.3104x over previous
import functools

import jax
import jax.numpy as jnp
from jax.experimental import pallas as pl

_EPS = 1e-5
_NUM_POINTS = (2048, 512, 128)
_RADII = ((0.2, 0.4, 0.8), (0.4, 0.8, 1.6), (1.6, 3.2, 4.8))
_NSAMPLES = ((32, 32, 64), (32, 32, 64), (32, 32, 32))


def _fps_body(npoint, n, xyzt_ref, idx_ref):
    X = xyzt_ref[...]
    b = X.shape[0]
    iota = jax.lax.broadcasted_iota(jnp.int32, (b, n), 1)

    def body(i, carry):
        dists, far = carry
        idx_ref[pl.ds(i, 1), :] = far[None, :]
        mask = (iota == far[:, None]).astype(jnp.float32)
        centroid = jnp.sum(X * mask[:, None, :], axis=2, keepdims=True)
        d = jnp.sum((X - centroid) ** 2, axis=1)
        dists = jnp.minimum(dists, d)
        m = jnp.max(dists, axis=1, keepdims=True)
        far = jnp.min(jnp.where(dists == m, iota, n), axis=1).astype(jnp.int32)
        return dists, far

    dists0 = jnp.full((b, n), 1e10, jnp.float32)
    far0 = jnp.zeros((b,), jnp.int32)
    jax.lax.fori_loop(0, npoint, body, (dists0, far0))


def _fps(xyz, npoint):
    b, n, _ = xyz.shape
    xyzt = jnp.transpose(xyz, (0, 2, 1))
    idx = pl.pallas_call(
        functools.partial(_fps_body, npoint, n),
        out_shape=jax.ShapeDtypeStruct((npoint, b), jnp.int32),
    )(xyzt)
    return idx.T


def _ball_query(d2, radius, nsample):
    n = d2.shape[-1]
    ar = jnp.arange(n, dtype=jnp.int32)
    cand = jnp.where(d2 < radius * radius, ar[None, None, :], n)
    cand = jnp.sort(cand, axis=-1)[:, :, :nsample]
    first = cand[:, :, :1]
    first = jnp.where(first >= n, 0, first)
    return jnp.where(cand >= n, first, cand).astype(jnp.int32)


def _conv_bn_relu(x, p, axes):
    if x.ndim == 4:
        y = jnp.einsum('bcsk,oc->bosk', x, p['W']) + p['b'][None, :, None, None]
        g = p['g'][None, :, None, None]
        be = p['be'][None, :, None, None]
    else:
        y = jnp.einsum('bcs,oc->bos', x, p['W']) + p['b'][None, :, None]
        g = p['g'][None, :, None]
        be = p['be'][None, :, None]
    m = jnp.mean(y, axis=axes, keepdims=True)
    v = jnp.var(y, axis=axes, keepdims=True)
    y = (y - m) / jnp.sqrt(v + _EPS)
    return jax.nn.relu(y * g + be)


def kernel(points, params):
    xyz = points[..., :3]
    feats = points[..., 3:]
    b = points.shape[0]
    bcol = jnp.arange(b)[:, None]
    bcube = jnp.arange(b)[:, None, None]
    indices = None
    for i, npoint in enumerate(_NUM_POINTS):
        fps_idx = _fps(xyz, npoint)
        indices = fps_idx if indices is None else jnp.take_along_axis(indices, fps_idx, axis=1)
        new_xyz = xyz[bcol, fps_idx]
        d2 = jnp.sum((jax.lax.stop_gradient(new_xyz)[:, :, None, :] -
                      jax.lax.stop_gradient(xyz)[:, None, :, :]) ** 2, axis=-1)
        branch_outs = []
        for r, radius in enumerate(_RADII[i]):
            idx = _ball_query(d2, radius, _NSAMPLES[i][r])
            gxyz = (xyz[bcube, idx] - new_xyz[:, :, None, :]) / radius
            gfeat = feats[bcube, idx]
            x = jnp.transpose(jnp.concatenate([gxyz, gfeat], axis=-1), (0, 3, 1, 2))
            for layer in params[i]['branches'][r]:
                x = _conv_bn_relu(x, layer, (0, 2, 3))
            branch_outs.append(jnp.max(x, axis=-1))
        cat = jnp.concatenate(branch_outs, axis=1)
        y = _conv_bn_relu(cat, params[i]['agg'], (0, 2))
        xyz = new_xyz
        feats = jnp.transpose(y, (0, 2, 1))
    return (xyz, jnp.transpose(feats, (0, 2, 1)), indices)
